# baseline (speedup 1.0000x reference)
import jax
import jax.numpy as jnp
from jax import lax
from jax.experimental import pallas as pl
from jax.experimental.pallas import tpu as pltpu

N, C = 1024, 100000
BR = 4096
NB = (C + BR - 1) // BR
LAST = NB - 1
ISSUE_STEPS = 16
RPS = N // ISSUE_STEPS
LOG2E = 1.4426950408889634


def _window_copies(tgt, cosw, phiw, cwin, pwin, sem_c, sem_p, i):
    t = tgt[i]
    i0 = pl.multiple_of(jnp.bitwise_and(i, -128), 128)
    cc = pltpu.make_async_copy(cosw.at[pl.ds(t, 1), pl.ds(i0, 128)],
                               cwin.at[pl.ds(i, 1)], sem_c)
    pc = pltpu.make_async_copy(phiw.at[pl.ds(t, 1), pl.ds(i0, 128)],
                               pwin.at[pl.ds(i, 1)], sem_p)
    return cc, pc


def _body(tgt, cos_ref, cosw, phiw, out_ref,
          m_ref, s_ref, cwin, pwin, sem_c, sem_p):
    k = pl.program_id(0)

    @pl.when(k == 0)
    def _init():
        m_ref[...] = jnp.full(m_ref.shape, -jnp.inf, jnp.float32)
        s_ref[...] = jnp.zeros(s_ref.shape, jnp.float32)

    @pl.when(k < ISSUE_STEPS)
    def _issue():
        def body(i, carry):
            cc, pc = _window_copies(tgt, cosw, phiw, cwin, pwin,
                                    sem_c, sem_p, i)
            cc.start()
            pc.start()
            return carry
        lax.fori_loop(k * RPS, (k + 1) * RPS, body, 0)

    SH = 32

    def update(strip_fn, nstrips):
        m = m_ref[...]

        def maxbody(c, acc):
            return jnp.maximum(
                acc, jnp.max(strip_fn(c), axis=0, keepdims=True))
        bm = lax.fori_loop(0, nstrips, maxbody,
                           jnp.full((1, N), -jnp.inf, jnp.float32))
        mn = jnp.maximum(m, bm)
        mnl = mn * LOG2E

        def sumbody(c, acc):
            return acc + jnp.sum(jnp.exp2(strip_fn(c) * LOG2E - mnl),
                                 axis=0, keepdims=True)
        bs = lax.fori_loop(0, nstrips, sumbody, jnp.zeros((1, N), jnp.float32))
        s_ref[...] = s_ref[...] * jnp.exp2(m * LOG2E - mnl) + bs
        m_ref[...] = mn

    @pl.when(k < LAST)
    def _full():
        update(lambda c: cos_ref[pl.ds(c * SH, SH), :], BR // SH)

    @pl.when(k == LAST)
    def _last():
        tail = C - LAST * BR

        def strip(c):
            blk = cos_ref[pl.ds(c * SH, SH), :]
            ids = lax.broadcasted_iota(jnp.int32, (SH, N), 0) + c * SH
            return jnp.where(ids < tail, blk, -jnp.inf)
        update(strip, (tail + SH - 1) // SH)

        def wbody(i, carry):
            cc, pc = _window_copies(tgt, cosw, phiw, cwin, pwin,
                                    sem_c, sem_p, i)
            cc.wait()
            pc.wait()
            return carry
        lax.fori_loop(0, N, wbody, 0)

        row = lax.broadcasted_iota(jnp.int32, (N, 128), 0)
        lanes = lax.broadcasted_iota(jnp.int32, (N, 128), 1)
        sel = lanes == jnp.bitwise_and(row, 127)
        cost = jnp.sum(jnp.where(sel, cwin[...], 0.0), axis=1, keepdims=True)
        phit = jnp.sum(jnp.where(sel, pwin[...], 0.0), axis=1, keepdims=True)
        cost = jnp.transpose(cost)
        phit = jnp.transpose(phit)

        m = m_ref[...]
        s = s_ref[...]
        mf = jnp.maximum(m, phit)
        z = s * jnp.exp(m - mf) - jnp.exp(cost - mf) + jnp.exp(phit - mf)
        nll = mf + jnp.log(z) - phit
        out_ref[...] = (jnp.sum(nll) * (1.0 / N)).reshape(1, 1)


_grid_spec = pltpu.PrefetchScalarGridSpec(
    num_scalar_prefetch=1,
    grid=(NB,),
    in_specs=[
        pl.BlockSpec((BR, N), lambda k, tgt: (k, 0)),
        pl.BlockSpec(memory_space=pl.ANY),
        pl.BlockSpec(memory_space=pl.ANY),
    ],
    out_specs=pl.BlockSpec((1, 1), lambda k, tgt: (0, 0)),
    scratch_shapes=[
        pltpu.VMEM((1, N), jnp.float32),
        pltpu.VMEM((1, N), jnp.float32),
        pltpu.VMEM((N, 128), jnp.float32),
        pltpu.VMEM((N, 128), jnp.float32),
        pltpu.SemaphoreType.DMA,
        pltpu.SemaphoreType.DMA,
    ],
)

_pass = pl.pallas_call(
    _body,
    grid_spec=_grid_spec,
    out_shape=jax.ShapeDtypeStruct((1, 1), jnp.float32),
    compiler_params=pltpu.CompilerParams(
        dimension_semantics=("arbitrary",),
    ),
)


@jax.jit
def _impl(cos, phi, target):
    cos_t = cos.T
    phi_t = phi.T
    loss = _pass(target, cos_t, cos_t, phi_t)
    return loss[0, 0]


def kernel(cos, phi, target):
    return _impl(cos, phi, target)


# speedup vs baseline: 1.4069x; 1.4069x over previous
import jax
import jax.numpy as jnp
from jax import lax
from jax.experimental import pallas as pl
from jax.experimental.pallas import tpu as pltpu

N, C = 1024, 100000
BR = 4096
NB = (C + BR - 1) // BR
LAST = NB - 1
ISSUE_STEPS = 16
RPS = N // ISSUE_STEPS
LOG2E = 1.4426950408889634


def _window_copies(tgt, cosw, phiw, cwin, pwin, sem_c, sem_p, i):
    t = tgt[i]
    i0 = pl.multiple_of(jnp.bitwise_and(i, -128), 128)
    cc = pltpu.make_async_copy(cosw.at[pl.ds(t, 1), pl.ds(i0, 128)],
                               cwin.at[pl.ds(i, 1)], sem_c)
    pc = pltpu.make_async_copy(phiw.at[pl.ds(t, 1), pl.ds(i0, 128)],
                               pwin.at[pl.ds(i, 1)], sem_p)
    return cc, pc


def _body(tgt, cos_ref, cosw, phiw, out_ref,
          m_ref, s_ref, cwin, pwin, sem_c, sem_p):
    k = pl.program_id(0)

    @pl.when(k == 0)
    def _init():
        m_ref[...] = jnp.full(m_ref.shape, -jnp.inf, jnp.float32)
        s_ref[...] = jnp.zeros(s_ref.shape, jnp.float32)

    @pl.when(k < ISSUE_STEPS)
    def _issue():
        def body(i, carry):
            cc, pc = _window_copies(tgt, cosw, phiw, cwin, pwin,
                                    sem_c, sem_p, i)
            cc.start()
            pc.start()
            return carry
        lax.fori_loop(k * RPS, (k + 1) * RPS, body, 0)

    def update(block):
        m = m_ref[...]
        bm = jnp.max(block, axis=0, keepdims=True)
        mn = jnp.maximum(m, bm)
        s_ref[...] = (s_ref[...] * jnp.exp(m - mn)
                      + jnp.sum(jnp.exp(block - mn), axis=0, keepdims=True))
        m_ref[...] = mn

    @pl.when(k < LAST)
    def _full():
        update(cos_ref[...])

    @pl.when(k == LAST)
    def _last():
        block = cos_ref[...]
        ids = lax.broadcasted_iota(jnp.int32, block.shape, 0) + k * BR
        update(jnp.where(ids < C, block, -jnp.inf))

        def wbody(i, carry):
            cc, pc = _window_copies(tgt, cosw, phiw, cwin, pwin,
                                    sem_c, sem_p, i)
            cc.wait()
            pc.wait()
            return carry
        lax.fori_loop(0, N, wbody, 0)

        row = lax.broadcasted_iota(jnp.int32, (N, 128), 0)
        lanes = lax.broadcasted_iota(jnp.int32, (N, 128), 1)
        sel = lanes == jnp.bitwise_and(row, 127)
        cost = jnp.sum(jnp.where(sel, cwin[...], 0.0), axis=1, keepdims=True)
        phit = jnp.sum(jnp.where(sel, pwin[...], 0.0), axis=1, keepdims=True)
        cost = jnp.transpose(cost)
        phit = jnp.transpose(phit)

        m = m_ref[...]
        s = s_ref[...]
        mf = jnp.maximum(m, phit)
        z = s * jnp.exp(m - mf) - jnp.exp(cost - mf) + jnp.exp(phit - mf)
        nll = mf + jnp.log(z) - phit
        out_ref[...] = (jnp.sum(nll) * (1.0 / N)).reshape(1, 1)


_grid_spec = pltpu.PrefetchScalarGridSpec(
    num_scalar_prefetch=1,
    grid=(NB,),
    in_specs=[
        pl.BlockSpec((BR, N), lambda k, tgt: (k, 0)),
        pl.BlockSpec(memory_space=pl.ANY),
        pl.BlockSpec(memory_space=pl.ANY),
    ],
    out_specs=pl.BlockSpec((1, 1), lambda k, tgt: (0, 0)),
    scratch_shapes=[
        pltpu.VMEM((1, N), jnp.float32),
        pltpu.VMEM((1, N), jnp.float32),
        pltpu.VMEM((N, 128), jnp.float32),
        pltpu.VMEM((N, 128), jnp.float32),
        pltpu.SemaphoreType.DMA,
        pltpu.SemaphoreType.DMA,
    ],
)

_pass = pl.pallas_call(
    _body,
    grid_spec=_grid_spec,
    out_shape=jax.ShapeDtypeStruct((1, 1), jnp.float32),
    compiler_params=pltpu.CompilerParams(
        dimension_semantics=("arbitrary",),
    ),
)


@jax.jit
def _impl(cos, phi, target):
    cos_t = cos.T
    phi_t = phi.T
    loss = _pass(target, cos_t, cos_t, phi_t)
    return loss[0, 0]


def kernel(cos, phi, target):
    return _impl(cos, phi, target)


# speedup vs baseline: 1.4075x; 1.0004x over previous
import jax
import jax.numpy as jnp
from jax import lax
from jax.experimental import pallas as pl
from jax.experimental.pallas import tpu as pltpu

N, C = 1024, 100000
BR = 4096
NB = (C + BR - 1) // BR
LAST = NB - 1
ISSUE_STEPS = 16
RPS = N // ISSUE_STEPS


def _window_copies(tgt, cosw, phiw, cwin, pwin, sem_c, sem_p, i):
    t = tgt[i]
    i0 = pl.multiple_of(jnp.bitwise_and(i, -128), 128)
    cc = pltpu.make_async_copy(cosw.at[pl.ds(t, 1), pl.ds(i0, 128)],
                               cwin.at[pl.ds(i, 1)], sem_c)
    pc = pltpu.make_async_copy(phiw.at[pl.ds(t, 1), pl.ds(i0, 128)],
                               pwin.at[pl.ds(i, 1)], sem_p)
    return cc, pc


def _body(tgt, cos_ref, cosw, phiw, out_ref,
          m_ref, s_ref, cwin, pwin, sem_c, sem_p):
    k = pl.program_id(0)

    @pl.when(k == 0)
    def _init():
        m_ref[...] = jnp.full(m_ref.shape, -jnp.inf, jnp.float32)
        s_ref[...] = jnp.zeros(s_ref.shape, jnp.float32)

    @pl.when(k < ISSUE_STEPS)
    def _issue():
        def body(i, carry):
            cc, pc = _window_copies(tgt, cosw, phiw, cwin, pwin,
                                    sem_c, sem_p, i)
            cc.start()
            pc.start()
            return carry
        lax.fori_loop(k * RPS, (k + 1) * RPS, body, 0)

    def update(block):
        m = m_ref[...]
        bm = jnp.max(block, axis=0, keepdims=True)
        mn = jnp.maximum(m, bm)
        s_ref[...] = (s_ref[...] * jnp.exp(m - mn)
                      + jnp.sum(jnp.exp(block - mn), axis=0, keepdims=True))
        m_ref[...] = mn

    @pl.when(k < LAST)
    def _full():
        update(cos_ref[...])

    @pl.when(k == LAST)
    def _last():
        block = cos_ref[...]
        ids = lax.broadcasted_iota(jnp.int32, block.shape, 0) + k * BR
        update(jnp.where(ids < C, block, -jnp.inf))

        def wbody(i, carry):
            cc, pc = _window_copies(tgt, cosw, phiw, cwin, pwin,
                                    sem_c, sem_p, i)
            cc.wait()
            pc.wait()
            return carry
        lax.fori_loop(0, N, wbody, 0)

        row = lax.broadcasted_iota(jnp.int32, (N, 128), 0)
        lanes = lax.broadcasted_iota(jnp.int32, (N, 128), 1)
        sel = lanes == jnp.bitwise_and(row, 127)
        cost = jnp.sum(jnp.where(sel, cwin[...], 0.0), axis=1, keepdims=True)
        phit = jnp.sum(jnp.where(sel, pwin[...], 0.0), axis=1, keepdims=True)
        cost = jnp.transpose(cost)
        phit = jnp.transpose(phit)

        m = m_ref[...]
        s = s_ref[...]
        mf = jnp.maximum(m, phit)
        z = s * jnp.exp(m - mf) - jnp.exp(cost - mf) + jnp.exp(phit - mf)
        nll = mf + jnp.log(z) - phit
        out_ref[...] = (jnp.sum(nll) * (1.0 / N)).reshape(1, 1)


_grid_spec = pltpu.PrefetchScalarGridSpec(
    num_scalar_prefetch=1,
    grid=(NB,),
    in_specs=[
        pl.BlockSpec((BR, N), lambda k, tgt: (k, 0)),
        pl.BlockSpec(memory_space=pl.ANY),
        pl.BlockSpec(memory_space=pl.ANY),
    ],
    out_specs=pl.BlockSpec((1, 1), lambda k, tgt: (0, 0)),
    scratch_shapes=[
        pltpu.VMEM((1, N), jnp.float32),
        pltpu.VMEM((1, N), jnp.float32),
        pltpu.VMEM((N, 128), jnp.float32),
        pltpu.VMEM((N, 128), jnp.float32),
        pltpu.SemaphoreType.DMA,
        pltpu.SemaphoreType.DMA,
    ],
)

_pass = pl.pallas_call(
    _body,
    grid_spec=_grid_spec,
    out_shape=jax.ShapeDtypeStruct((1, 1), jnp.float32),
    compiler_params=pltpu.CompilerParams(
        dimension_semantics=("arbitrary",),
    ),
)


@jax.jit
def _impl(cos, phi, target):
    cos_t = cos.T
    phi_t = phi.T
    loss = _pass(target, cos_t, cos_t, phi_t)
    return loss[0, 0]


def kernel(cos, phi, target):
    return _impl(cos, phi, target)
